# baseline (speedup 1.0000x reference)
import functools

import jax
import jax.numpy as jnp
from jax import lax
from jax.experimental import pallas as pl
from jax.experimental.pallas import tpu as pltpu
from jax.experimental.pallas import tpu_sc as plsc


def _y_body(w_ref, x_ref, y_ref, wacc):
    i = pl.program_id(0)
    r = pl.program_id(1)

    @pl.when(r == 0)
    def _():
        prev = jnp.where(i == 0, jnp.zeros_like(wacc[...]), wacc[...])
        wacc[...] = prev + w_ref[0]

    y_ref[0] = jnp.dot(x_ref[...].astype(jnp.bfloat16),
                       wacc[...].astype(jnp.bfloat16),
                       preferred_element_type=jnp.float32)


def _compute_y(x, weights):
    s, d_in, d_out = weights.shape
    n = x.shape[0]
    br = 1000
    return pl.pallas_call(
        _y_body,
        grid=(s, n // br),
        in_specs=[
            pl.BlockSpec((1, d_in, d_out), lambda i, r: (i, 0, 0)),
            pl.BlockSpec((br, d_in), lambda i, r: (r, 0)),
        ],
        out_specs=pl.BlockSpec((1, br, d_out), lambda i, r: (i, r, 0)),
        out_shape=jax.ShapeDtypeStruct((s, n, d_out), jnp.float32),
        scratch_shapes=[pltpu.VMEM((d_in, d_out), jnp.float32)],
    )(weights, x)


_LANES = 16
_HALF = 128
_CH = 80
_WB = 80


def _sc_agg(y_flat, gidx, sidx, vals, n_dst, n_src):
    s_sup, e_edges = gidx.shape
    n_tiles = 16
    epc = e_edges // n_tiles
    n_chunks = epc // _CH
    gidx = gidx.reshape(s_sup, n_tiles, n_chunks, _CH)
    sidx = sidx.reshape(s_sup, n_tiles, n_chunks, _CH)
    vals = vals.reshape(s_sup, n_tiles, n_chunks, _CH)
    row_chunks = n_dst // _WB
    wb_iters = -(-row_chunks // n_tiles)

    mesh = plsc.VectorSubcoreMesh(core_axis_name="c", subcore_axis_name="s")

    @functools.partial(
        pl.kernel,
        mesh=mesh,
        out_type=jax.ShapeDtypeStruct((n_dst, 2 * _HALF), jnp.float32),
        scratch_types=[
            pltpu.VMEM((n_chunks, _CH), jnp.int32),
            pltpu.VMEM((n_chunks, _CH), jnp.int32),
            pltpu.VMEM((n_chunks, _CH), jnp.float32),
            pltpu.VMEM((_CH, _HALF), jnp.float32),
            pltpu.VMEM((_CH, _HALF), jnp.float32),
            pltpu.VMEM((_CH, _HALF), jnp.float32),
            pltpu.VMEM_SHARED((n_dst, _HALF), jnp.float32),
            pltpu.SemaphoreType.DMA,
            pltpu.SemaphoreType.DMA,
            pltpu.SemaphoreType.DMA,
            pltpu.SemaphoreType.DMA,
            pltpu.SemaphoreType.DMA,
            pltpu.SemaphoreType.DMA,
        ],
    )
    def agg(y_hbm, g_hbm, s_hbm, v_hbm, out_hbm, gbuf, sbuf, vbuf, rb0, rb1,
            rb2, acc, gs0, gs1, gs2, ss0, ss1, ss2):
        wbuf = rb0
        c = lax.axis_index("c")
        s = lax.axis_index("s")

        def zero_body(e, carry):
            for q in range(_HALF // _LANES):
                wbuf[e, pl.ds(q * _LANES, _LANES)] = jnp.zeros(
                    (_LANES,), jnp.float32)
            return carry

        lax.fori_loop(0, _WB, zero_body, 0)

        def zero_chunk(j, carry):
            idx = s + j * n_tiles

            @pl.when(idx < row_chunks)
            def _():
                pltpu.sync_copy(wbuf, acc.at[pl.ds(idx * _WB, _WB)])

            return carry

        lax.fori_loop(0, wb_iters, zero_chunk, 0)
        plsc.subcore_barrier()

        bufs = (rb0, rb1, rb2)
        gsems = (gs0, gs1, gs2)
        ssems = (ss0, ss1, ss2)

        def start_gather(k, rb, sem):
            pltpu.async_copy(y_hbm.at[gbuf.at[k]], rb, sem)

        def wait_gather(rb, sem):
            pltpu.make_async_copy(y_hbm.at[pl.ds(0, _CH)], rb, sem).wait()

        def start_scatter(k, rb, sem):
            pltpu.async_copy(rb, acc.at[sbuf.at[k]], sem, add=True)

        def wait_scatter(rb, sem):
            pltpu.make_async_copy(rb, acc.at[pl.ds(0, _CH)], sem).wait()

        def scale(k, rb):
            def scale_g(g, carry2):
                vv = vbuf[k, pl.ds(g * _LANES, _LANES)]
                for j in range(_LANES):
                    e = g * _LANES + j
                    v = vv[j]
                    for q in range(_HALF // _LANES):
                        sl = pl.ds(q * _LANES, _LANES)
                        rb[e, sl] = rb[e, sl] * v
                return carry2

            lax.fori_loop(0, _CH // _LANES, scale_g, 0)

        for i in range(s_sup):
            base = jnp.int32(i * 2 * n_src) + c

            pltpu.sync_copy(g_hbm.at[i, s], gbuf)
            pltpu.sync_copy(s_hbm.at[i, s], sbuf)
            pltpu.sync_copy(v_hbm.at[i, s], vbuf)

            def gfix(k, carry, base=base):
                for m in range(_CH // _LANES):
                    sl = pl.ds(m * _LANES, _LANES)
                    gbuf[k, sl] = gbuf[k, sl] * 2 + base
                return carry

            lax.fori_loop(0, n_chunks, gfix, 0)

            start_gather(0, bufs[0], gsems[0])
            start_gather(1, bufs[1], gsems[1])

            def tri(t, carry):
                for off in range(3):
                    k = 3 * t + off
                    nxt = (off + 2) % 3

                    @pl.when(k < n_chunks)
                    def _(k=k, off=off, nxt=nxt):
                        wait_gather(bufs[off], gsems[off])
                        scale(k, bufs[off])
                        start_scatter(k, bufs[off], ssems[off])

                        @pl.when(k + 2 < n_chunks)
                        def _():
                            @pl.when(k >= 1)
                            def _():
                                wait_scatter(bufs[nxt], ssems[nxt])

                            start_gather(k + 2, bufs[nxt], gsems[nxt])

                return carry

            lax.fori_loop(0, -(-n_chunks // 3), tri, 0)

            for j in range(min(3, n_chunks)):
                k = n_chunks - 1 - j
                wait_scatter(bufs[k % 3], ssems[k % 3])

        plsc.subcore_barrier()

        def wb_chunk(j, carry):
            idx = s + j * n_tiles

            @pl.when(idx < row_chunks)
            def _():
                r0 = idx * _WB
                pltpu.sync_copy(acc.at[pl.ds(r0, _WB)], wbuf)

                def relu_body(e, carry2):
                    for q in range(_HALF // _LANES):
                        sl = pl.ds(q * _LANES, _LANES)
                        wbuf[e, sl] = jnp.maximum(wbuf[e, sl], 0.0)
                    return carry2

                lax.fori_loop(0, _WB, relu_body, 0)
                pltpu.sync_copy(wbuf, out_hbm.at[pl.ds(r0, _WB),
                                                 pl.ds(c * _HALF, _HALF)])

            return carry

        lax.fori_loop(0, wb_iters, wb_chunk, 0)

    return agg(y_flat, gidx, sidx, vals)


def kernel(x_u, x_v, sup_vals, weights_u, weights_v, sup_rows, sup_cols):
    n_u = x_u.shape[0]
    n_v = x_v.shape[0]
    rows = sup_rows.astype(jnp.int32)
    cols = sup_cols.astype(jnp.int32)
    vals = sup_vals.astype(jnp.float32)

    y_v = _compute_y(x_v, weights_v)
    y_v2 = y_v.reshape(-1, _HALF)
    z_u = _sc_agg(y_v2, cols, rows, vals, n_u, n_v)
    y_u = _compute_y(x_u, weights_u)
    y_u2 = y_u.reshape(-1, _HALF)
    z_v = _sc_agg(y_u2, rows, cols, vals, n_v, n_u)
    return z_u, z_v


# speedup vs baseline: 1.0011x; 1.0011x over previous
import functools

import jax
import jax.numpy as jnp
from jax import lax
from jax.experimental import pallas as pl
from jax.experimental.pallas import tpu as pltpu
from jax.experimental.pallas import tpu_sc as plsc


def _y_body(w_ref, x_ref, y_ref, wacc):
    i = pl.program_id(0)
    r = pl.program_id(1)

    @pl.when(r == 0)
    def _():
        prev = jnp.where(i == 0, jnp.zeros_like(wacc[...]), wacc[...])
        wacc[...] = prev + w_ref[0]

    y_ref[0] = jnp.dot(x_ref[...], wacc[...], preferred_element_type=jnp.float32)


def _compute_y(x, weights):
    s, d_in, d_out = weights.shape
    n = x.shape[0]
    br = 1000
    return pl.pallas_call(
        _y_body,
        grid=(s, n // br),
        in_specs=[
            pl.BlockSpec((1, d_in, d_out), lambda i, r: (i, 0, 0)),
            pl.BlockSpec((br, d_in), lambda i, r: (r, 0)),
        ],
        out_specs=pl.BlockSpec((1, br, d_out), lambda i, r: (i, r, 0)),
        out_shape=jax.ShapeDtypeStruct((s, n, d_out), jnp.float32),
        scratch_shapes=[pltpu.VMEM((d_in, d_out), jnp.float32)],
    )(weights, x)


_LANES = 16
_HALF = 128
_CH = 80
_WB = 80


def _sc_agg(y_flat, gidx, sidx, vals, n_dst, n_src):
    s_sup, e_edges = gidx.shape
    n_tiles = 16
    epc = e_edges // n_tiles
    n_chunks = epc // _CH
    gidx = gidx.reshape(s_sup, n_tiles, n_chunks, _CH)
    sidx = sidx.reshape(s_sup, n_tiles, n_chunks, _CH)
    vals = vals.reshape(s_sup, n_tiles, n_chunks, _CH)
    row_chunks = n_dst // _WB
    wb_iters = -(-row_chunks // n_tiles)

    mesh = plsc.VectorSubcoreMesh(core_axis_name="c", subcore_axis_name="s")

    @functools.partial(
        pl.kernel,
        mesh=mesh,
        out_type=jax.ShapeDtypeStruct((n_dst, 2 * _HALF), jnp.float32),
        scratch_types=[
            pltpu.VMEM((n_chunks, _CH), jnp.int32),
            pltpu.VMEM((n_chunks, _CH), jnp.int32),
            pltpu.VMEM((n_chunks, _CH), jnp.float32),
            pltpu.VMEM((_CH, _HALF), jnp.float32),
            pltpu.VMEM((_CH, _HALF), jnp.float32),
            pltpu.VMEM((_CH, _HALF), jnp.float32),
            pltpu.VMEM_SHARED((n_dst, _HALF), jnp.float32),
            pltpu.SemaphoreType.DMA,
            pltpu.SemaphoreType.DMA,
            pltpu.SemaphoreType.DMA,
            pltpu.SemaphoreType.DMA,
            pltpu.SemaphoreType.DMA,
            pltpu.SemaphoreType.DMA,
        ],
    )
    def agg(y_hbm, g_hbm, s_hbm, v_hbm, out_hbm, gbuf, sbuf, vbuf, rb0, rb1,
            rb2, acc, gs0, gs1, gs2, ss0, ss1, ss2):
        wbuf = rb0
        c = lax.axis_index("c")
        s = lax.axis_index("s")

        def zero_body(e, carry):
            for q in range(_HALF // _LANES):
                wbuf[e, pl.ds(q * _LANES, _LANES)] = jnp.zeros(
                    (_LANES,), jnp.float32)
            return carry

        lax.fori_loop(0, _WB, zero_body, 0)

        def zero_chunk(j, carry):
            idx = s + j * n_tiles

            @pl.when(idx < row_chunks)
            def _():
                pltpu.sync_copy(wbuf, acc.at[pl.ds(idx * _WB, _WB)])

            return carry

        lax.fori_loop(0, wb_iters, zero_chunk, 0)
        plsc.subcore_barrier()

        bufs = (rb0, rb1, rb2)
        gsems = (gs0, gs1, gs2)
        ssems = (ss0, ss1, ss2)

        def start_gather(k, rb, sem):
            pltpu.async_copy(y_hbm.at[gbuf.at[k]], rb, sem)

        def wait_gather(rb, sem):
            pltpu.make_async_copy(y_hbm.at[pl.ds(0, _CH)], rb, sem).wait()

        def start_scatter(k, rb, sem):
            pltpu.async_copy(rb, acc.at[sbuf.at[k]], sem, add=True)

        def wait_scatter(rb, sem):
            pltpu.make_async_copy(rb, acc.at[pl.ds(0, _CH)], sem).wait()

        def scale(k, rb):
            def scale_g(g, carry2):
                vv = vbuf[k, pl.ds(g * _LANES, _LANES)]
                for j in range(_LANES):
                    e = g * _LANES + j
                    v = vv[j]
                    for q in range(_HALF // _LANES):
                        sl = pl.ds(q * _LANES, _LANES)
                        rb[e, sl] = rb[e, sl] * v
                return carry2

            lax.fori_loop(0, _CH // _LANES, scale_g, 0)

        for i in range(s_sup):
            base = jnp.int32(i * 2 * n_src) + c

            pltpu.sync_copy(g_hbm.at[i, s], gbuf)
            pltpu.sync_copy(s_hbm.at[i, s], sbuf)
            pltpu.sync_copy(v_hbm.at[i, s], vbuf)

            def gfix(k, carry, base=base):
                for m in range(_CH // _LANES):
                    sl = pl.ds(m * _LANES, _LANES)
                    gbuf[k, sl] = gbuf[k, sl] * 2 + base
                return carry

            lax.fori_loop(0, n_chunks, gfix, 0)

            start_gather(0, bufs[0], gsems[0])
            start_gather(1, bufs[1], gsems[1])

            def tri(t, carry):
                for off in range(3):
                    k = 3 * t + off
                    nxt = (off + 2) % 3

                    @pl.when(k < n_chunks)
                    def _(k=k, off=off, nxt=nxt):
                        wait_gather(bufs[off], gsems[off])
                        scale(k, bufs[off])
                        start_scatter(k, bufs[off], ssems[off])

                        @pl.when(k + 2 < n_chunks)
                        def _():
                            @pl.when(k >= 1)
                            def _():
                                wait_scatter(bufs[nxt], ssems[nxt])

                            start_gather(k + 2, bufs[nxt], gsems[nxt])

                return carry

            lax.fori_loop(0, -(-n_chunks // 3), tri, 0)

            for j in range(min(3, n_chunks)):
                k = n_chunks - 1 - j
                wait_scatter(bufs[k % 3], ssems[k % 3])

        plsc.subcore_barrier()

        def wb_chunk(j, carry):
            idx = s + j * n_tiles

            @pl.when(idx < row_chunks)
            def _():
                r0 = idx * _WB
                pltpu.sync_copy(acc.at[pl.ds(r0, _WB)], wbuf)

                def relu_body(e, carry2):
                    for q in range(_HALF // _LANES):
                        sl = pl.ds(q * _LANES, _LANES)
                        wbuf[e, sl] = jnp.maximum(wbuf[e, sl], 0.0)
                    return carry2

                lax.fori_loop(0, _WB, relu_body, 0)
                pltpu.sync_copy(wbuf, out_hbm.at[pl.ds(r0, _WB),
                                                 pl.ds(c * _HALF, _HALF)])

            return carry

        lax.fori_loop(0, wb_iters, wb_chunk, 0)

    return agg(y_flat, gidx, sidx, vals)


def kernel(x_u, x_v, sup_vals, weights_u, weights_v, sup_rows, sup_cols):
    n_u = x_u.shape[0]
    n_v = x_v.shape[0]
    rows = sup_rows.astype(jnp.int32)
    cols = sup_cols.astype(jnp.int32)
    vals = sup_vals.astype(jnp.float32)

    y_v = _compute_y(x_v, weights_v)
    y_v2 = y_v.reshape(-1, _HALF)
    z_u = _sc_agg(y_v2, cols, rows, vals, n_u, n_v)
    y_u = _compute_y(x_u, weights_u)
    y_u2 = y_u.reshape(-1, _HALF)
    z_v = _sc_agg(y_u2, rows, cols, vals, n_v, n_u)
    return z_u, z_v


# speedup vs baseline: 1.1638x; 1.1625x over previous
import functools

import jax
import jax.numpy as jnp
from jax import lax
from jax.experimental import pallas as pl
from jax.experimental.pallas import tpu as pltpu
from jax.experimental.pallas import tpu_sc as plsc


def _y_body(w_ref, x_ref, y_ref, wacc):
    i = pl.program_id(0)
    r = pl.program_id(1)

    @pl.when(r == 0)
    def _():
        prev = jnp.where(i == 0, jnp.zeros_like(wacc[...]), wacc[...])
        wacc[...] = prev + w_ref[0]

    res = jnp.dot(x_ref[...], wacc[...], preferred_element_type=jnp.float32)
    d_half = res.shape[1] // 2
    y_ref[0, 0] = res[:, :d_half]
    y_ref[0, 1] = res[:, d_half:]


def _compute_y(x, weights):
    s, d_in, d_out = weights.shape
    n = x.shape[0]
    br = 1000
    return pl.pallas_call(
        _y_body,
        grid=(s, n // br),
        in_specs=[
            pl.BlockSpec((1, d_in, d_out), lambda i, r: (i, 0, 0)),
            pl.BlockSpec((br, d_in), lambda i, r: (r, 0)),
        ],
        out_specs=pl.BlockSpec((1, 2, br, d_out // 2),
                               lambda i, r: (i, 0, r, 0)),
        out_shape=jax.ShapeDtypeStruct((s, 2, n, d_out // 2), jnp.float32),
        scratch_shapes=[pltpu.VMEM((d_in, d_out), jnp.float32)],
    )(weights, x)


_LANES = 16
_HALF = 128
_CH = 80
_WB = 80


def _sc_agg(y_flat, gidx, sidx, vals, n_dst, n_src):
    s_sup, e_edges = gidx.shape
    n_tiles = 16
    epc = e_edges // n_tiles
    n_chunks = epc // _CH
    gidx = gidx.reshape(s_sup, n_tiles, n_chunks, _CH)
    sidx = sidx.reshape(s_sup, n_tiles, n_chunks, _CH)
    vals = vals.reshape(s_sup, n_tiles, n_chunks, _CH)
    row_chunks = n_dst // _WB
    wb_iters = -(-row_chunks // n_tiles)

    mesh = plsc.VectorSubcoreMesh(core_axis_name="c", subcore_axis_name="s")

    @functools.partial(
        pl.kernel,
        mesh=mesh,
        out_type=jax.ShapeDtypeStruct((n_dst, 2 * _HALF), jnp.float32),
        scratch_types=[
            pltpu.VMEM((n_chunks, _CH), jnp.int32),
            pltpu.VMEM((n_chunks, _CH), jnp.int32),
            pltpu.VMEM((n_chunks, _CH), jnp.float32),
            pltpu.VMEM((_CH, _HALF), jnp.float32),
            pltpu.VMEM((_CH, _HALF), jnp.float32),
            pltpu.VMEM((_CH, _HALF), jnp.float32),
            pltpu.VMEM_SHARED((n_dst, _HALF), jnp.float32),
            pltpu.SemaphoreType.DMA,
            pltpu.SemaphoreType.DMA,
            pltpu.SemaphoreType.DMA,
            pltpu.SemaphoreType.DMA,
            pltpu.SemaphoreType.DMA,
            pltpu.SemaphoreType.DMA,
        ],
    )
    def agg(y_hbm, g_hbm, s_hbm, v_hbm, out_hbm, gbuf, sbuf, vbuf, rb0, rb1,
            rb2, acc, gs0, gs1, gs2, ss0, ss1, ss2):
        wbuf = rb0
        c = lax.axis_index("c")
        s = lax.axis_index("s")

        def zero_body(e, carry):
            for q in range(_HALF // _LANES):
                wbuf[e, pl.ds(q * _LANES, _LANES)] = jnp.zeros(
                    (_LANES,), jnp.float32)
            return carry

        lax.fori_loop(0, _WB, zero_body, 0)

        def zero_chunk(j, carry):
            idx = s + j * n_tiles

            @pl.when(idx < row_chunks)
            def _():
                pltpu.sync_copy(wbuf, acc.at[pl.ds(idx * _WB, _WB)])

            return carry

        lax.fori_loop(0, wb_iters, zero_chunk, 0)
        plsc.subcore_barrier()

        bufs = (rb0, rb1, rb2)
        gsems = (gs0, gs1, gs2)
        ssems = (ss0, ss1, ss2)

        def start_gather(k, rb, sem):
            pltpu.async_copy(y_hbm.at[gbuf.at[k]], rb, sem)

        def wait_gather(rb, sem):
            pltpu.make_async_copy(y_hbm.at[pl.ds(0, _CH)], rb, sem).wait()

        def start_scatter(k, rb, sem):
            pltpu.async_copy(rb, acc.at[sbuf.at[k]], sem, add=True)

        def wait_scatter(rb, sem):
            pltpu.make_async_copy(rb, acc.at[pl.ds(0, _CH)], sem).wait()

        def scale(k, rb):
            def scale_g(g, carry2):
                vv = vbuf[k, pl.ds(g * _LANES, _LANES)]
                for j in range(_LANES):
                    e = g * _LANES + j
                    v = vv[j]
                    for q in range(_HALF // _LANES):
                        sl = pl.ds(q * _LANES, _LANES)
                        rb[e, sl] = rb[e, sl] * v
                return carry2

            lax.fori_loop(0, _CH // _LANES, scale_g, 0)

        for i in range(s_sup):
            base = (jnp.int32(i * 2) + c) * n_src

            pltpu.sync_copy(g_hbm.at[i, s], gbuf)
            pltpu.sync_copy(s_hbm.at[i, s], sbuf)
            pltpu.sync_copy(v_hbm.at[i, s], vbuf)

            def gfix(k, carry, base=base):
                for m in range(_CH // _LANES):
                    sl = pl.ds(m * _LANES, _LANES)
                    gbuf[k, sl] = gbuf[k, sl] + base
                return carry

            lax.fori_loop(0, n_chunks, gfix, 0)

            start_gather(0, bufs[0], gsems[0])
            start_gather(1, bufs[1], gsems[1])

            def tri(t, carry):
                for off in range(3):
                    k = 3 * t + off
                    nxt = (off + 2) % 3

                    @pl.when(k < n_chunks)
                    def _(k=k, off=off, nxt=nxt):
                        wait_gather(bufs[off], gsems[off])
                        scale(k, bufs[off])
                        start_scatter(k, bufs[off], ssems[off])

                        @pl.when(k + 2 < n_chunks)
                        def _():
                            @pl.when(k >= 1)
                            def _():
                                wait_scatter(bufs[nxt], ssems[nxt])

                            start_gather(k + 2, bufs[nxt], gsems[nxt])

                return carry

            lax.fori_loop(0, -(-n_chunks // 3), tri, 0)

            for j in range(min(3, n_chunks)):
                k = n_chunks - 1 - j
                wait_scatter(bufs[k % 3], ssems[k % 3])

        plsc.subcore_barrier()

        def wb_chunk(j, carry):
            idx = s + j * n_tiles

            @pl.when(idx < row_chunks)
            def _():
                r0 = idx * _WB
                pltpu.sync_copy(acc.at[pl.ds(r0, _WB)], wbuf)

                def relu_body(e, carry2):
                    for q in range(_HALF // _LANES):
                        sl = pl.ds(q * _LANES, _LANES)
                        wbuf[e, sl] = jnp.maximum(wbuf[e, sl], 0.0)
                    return carry2

                lax.fori_loop(0, _WB, relu_body, 0)
                pltpu.sync_copy(wbuf, out_hbm.at[pl.ds(r0, _WB),
                                                 pl.ds(c * _HALF, _HALF)])

            return carry

        lax.fori_loop(0, wb_iters, wb_chunk, 0)

    return agg(y_flat, gidx, sidx, vals)


def kernel(x_u, x_v, sup_vals, weights_u, weights_v, sup_rows, sup_cols):
    n_u = x_u.shape[0]
    n_v = x_v.shape[0]
    rows = sup_rows.astype(jnp.int32)
    cols = sup_cols.astype(jnp.int32)
    vals = sup_vals.astype(jnp.float32)

    y_v = _compute_y(x_v, weights_v)
    y_v2 = y_v.reshape(-1, _HALF)
    z_u = _sc_agg(y_v2, cols, rows, vals, n_u, n_v)
    y_u = _compute_y(x_u, weights_u)
    y_u2 = y_u.reshape(-1, _HALF)
    z_v = _sc_agg(y_u2, rows, cols, vals, n_v, n_u)
    return z_u, z_v


# speedup vs baseline: 1.2403x; 1.0657x over previous
import functools

import jax
import jax.numpy as jnp
from jax import lax
from jax.experimental import pallas as pl
from jax.experimental.pallas import tpu as pltpu
from jax.experimental.pallas import tpu_sc as plsc


def _y_body(w_ref, x_ref, y_ref, wacc):
    i = pl.program_id(0)
    r = pl.program_id(1)

    @pl.when(r == 0)
    def _():
        prev = jnp.where(i == 0, jnp.zeros_like(wacc[...]), wacc[...])
        wacc[...] = prev + w_ref[0]

    res = jnp.dot(x_ref[...], wacc[...], preferred_element_type=jnp.float32)
    d_half = res.shape[1] // 2
    y_ref[0, 0] = res[:, :d_half]
    y_ref[0, 1] = res[:, d_half:]


def _compute_y(x, weights):
    s, d_in, d_out = weights.shape
    n = x.shape[0]
    br = 1000
    return pl.pallas_call(
        _y_body,
        grid=(s, n // br),
        in_specs=[
            pl.BlockSpec((1, d_in, d_out), lambda i, r: (i, 0, 0)),
            pl.BlockSpec((br, d_in), lambda i, r: (r, 0)),
        ],
        out_specs=pl.BlockSpec((1, 2, br, d_out // 2),
                               lambda i, r: (i, 0, r, 0)),
        out_shape=jax.ShapeDtypeStruct((s, 2, n, d_out // 2), jnp.float32),
        scratch_shapes=[pltpu.VMEM((d_in, d_out), jnp.float32)],
    )(weights, x)


_LANES = 16
_HALF = 128
_CH = 80
_WB = 80


def _sc_agg(y_flat, gidx, sidx, vals, n_dst, n_src):
    s_sup, e_edges = gidx.shape
    n_tiles = 16
    epc = e_edges // n_tiles
    n_chunks = epc // _CH
    gidx = gidx.reshape(-1)
    sidx = sidx.reshape(s_sup, n_tiles, n_chunks, _CH)
    vals = vals.reshape(-1)
    row_chunks = n_dst // _WB
    wb_iters = -(-row_chunks // n_tiles)

    mesh = plsc.VectorSubcoreMesh(core_axis_name="c", subcore_axis_name="s")

    @functools.partial(
        pl.kernel,
        mesh=mesh,
        out_type=jax.ShapeDtypeStruct((n_dst, 2 * _HALF), jnp.float32),
        scratch_types=[
            pltpu.VMEM((2 * epc,), jnp.int32),
            pltpu.VMEM((2, n_chunks, _CH), jnp.int32),
            pltpu.VMEM((2 * epc,), jnp.float32),
            pltpu.VMEM((_CH, _HALF), jnp.float32),
            pltpu.VMEM((_CH, _HALF), jnp.float32),
            pltpu.VMEM((_CH, _HALF), jnp.float32),
            pltpu.VMEM_SHARED((n_dst, _HALF), jnp.float32),
            pltpu.SemaphoreType.DMA,
            pltpu.SemaphoreType.DMA,
            pltpu.SemaphoreType.DMA,
            pltpu.SemaphoreType.DMA,
            pltpu.SemaphoreType.DMA,
            pltpu.SemaphoreType.DMA,
            pltpu.SemaphoreType.DMA,
        ],
    )
    def agg(y_hbm, gf_hbm, s_hbm, vf_hbm, out_hbm, gbuf, sbuf, vbuf, rb0, rb1,
            rb2, acc, gs0, gs1, gs2, ss0, ss1, ss2, stg):
        c = lax.axis_index("c")
        s = lax.axis_index("s")

        def stage(i, slot):
            e0 = (i * n_tiles + s) * epc
            pltpu.async_copy(gf_hbm.at[pl.ds(e0, epc)],
                             gbuf.at[pl.ds(slot * epc, epc)], stg)
            pltpu.async_copy(s_hbm.at[i, s], sbuf.at[slot], stg)
            pltpu.async_copy(vf_hbm.at[pl.ds(e0, epc)],
                             vbuf.at[pl.ds(slot * epc, epc)], stg)

        def stage_wait(slot):
            pltpu.make_async_copy(gf_hbm.at[pl.ds(0, epc)],
                                  gbuf.at[pl.ds(slot * epc, epc)], stg).wait()
            pltpu.make_async_copy(s_hbm.at[0, 0], sbuf.at[slot], stg).wait()
            pltpu.make_async_copy(vf_hbm.at[pl.ds(0, epc)],
                                  vbuf.at[pl.ds(slot * epc, epc)], stg).wait()

        stage(0, 0)

        def zero_body(e, carry):
            for q in range(_HALF // _LANES):
                rb0[e, pl.ds(q * _LANES, _LANES)] = jnp.zeros(
                    (_LANES,), jnp.float32)
            return carry

        lax.fori_loop(0, _WB, zero_body, 0)

        def zero_chunk(j, carry):
            idx = s + j * n_tiles

            @pl.when(idx < row_chunks)
            def _():
                pltpu.async_copy(rb0, acc.at[pl.ds(idx * _WB, _WB)], ss0)

            return carry

        lax.fori_loop(0, wb_iters, zero_chunk, 0)

        def zero_drain(j, carry):
            idx = s + j * n_tiles

            @pl.when(idx < row_chunks)
            def _():
                pltpu.make_async_copy(rb0, acc.at[pl.ds(0, _WB)], ss0).wait()

            return carry

        lax.fori_loop(0, wb_iters, zero_drain, 0)
        plsc.subcore_barrier()

        bufs = (rb0, rb1, rb2)
        gsems = (gs0, gs1, gs2)
        ssems = (ss0, ss1, ss2)

        def wait_gather(rb, sem):
            pltpu.make_async_copy(y_hbm.at[pl.ds(0, _CH)], rb, sem).wait()

        def wait_scatter(rb, sem):
            pltpu.make_async_copy(rb, acc.at[pl.ds(0, _CH)], sem).wait()

        for i in range(s_sup):
            slot = i % 2
            base = (jnp.int32(i * 2) + c) * n_src

            def start_gather(k, rb, sem, slot=slot):
                pltpu.async_copy(
                    y_hbm.at[gbuf.at[pl.ds(slot * epc + k * _CH, _CH)]],
                    rb, sem)

            def start_scatter(k, rb, sem, slot=slot):
                pltpu.async_copy(rb, acc.at[sbuf.at[slot, k]], sem, add=True)

            def scale(k, rb, slot=slot):
                def scale_g(g, carry2):
                    vv = vbuf[pl.ds(slot * epc + k * _CH + g * _LANES,
                                    _LANES)]
                    for j in range(_LANES):
                        e = g * _LANES + j
                        v = vv[j]
                        for q in range(_HALF // _LANES):
                            sl = pl.ds(q * _LANES, _LANES)
                            rb[e, sl] = rb[e, sl] * v
                    return carry2

                lax.fori_loop(0, _CH // _LANES, scale_g, 0)

            stage_wait(slot)
            if i + 1 < s_sup:
                stage(i + 1, (i + 1) % 2)

            def gfix(k, carry, base=base, slot=slot):
                sl = pl.ds(slot * epc + k * _LANES, _LANES)
                gbuf[sl] = gbuf[sl] + base
                return carry

            lax.fori_loop(0, epc // _LANES, gfix, 0)

            start_gather(0, bufs[0], gsems[0])
            start_gather(1, bufs[1], gsems[1])

            def tri(t, carry):
                for off in range(3):
                    k = 3 * t + off
                    nxt = (off + 2) % 3

                    @pl.when(k < n_chunks)
                    def _(k=k, off=off, nxt=nxt):
                        wait_gather(bufs[off], gsems[off])
                        scale(k, bufs[off])
                        start_scatter(k, bufs[off], ssems[off])

                        @pl.when(k + 2 < n_chunks)
                        def _():
                            @pl.when(k >= 1)
                            def _():
                                wait_scatter(bufs[nxt], ssems[nxt])

                            start_gather(k + 2, bufs[nxt], gsems[nxt])

                return carry

            lax.fori_loop(0, -(-n_chunks // 3), tri, 0)

            for j in range(min(3, n_chunks)):
                k = n_chunks - 1 - j
                wait_scatter(bufs[k % 3], ssems[k % 3])

        plsc.subcore_barrier()

        def wb_in(idx, rb, sem):
            pltpu.async_copy(acc.at[pl.ds(idx * _WB, _WB)], rb, sem)

        def wb_in_wait(rb, sem):
            pltpu.make_async_copy(acc.at[pl.ds(0, _WB)], rb, sem).wait()

        def relu_buf(rb):
            def relu_body(e, carry2):
                for q in range(_HALF // _LANES):
                    sl = pl.ds(q * _LANES, _LANES)
                    rb[e, sl] = jnp.maximum(rb[e, sl], 0.0)
                return carry2

            lax.fori_loop(0, _WB, relu_body, 0)

        def wb_out(idx, rb):
            pltpu.sync_copy(rb, out_hbm.at[pl.ds(idx * _WB, _WB),
                                           pl.ds(c * _HALF, _HALF)])

        @pl.when(s < row_chunks)
        def _():
            wb_in(s, rb0, gs0)

        def wpair(p, carry):
            j0 = 2 * p
            j1 = 2 * p + 1
            i0 = s + j0 * n_tiles
            i1 = s + j1 * n_tiles

            @pl.when(i1 < row_chunks)
            def _():
                wb_in(i1, rb1, gs1)

            @pl.when(i0 < row_chunks)
            def _():
                wb_in_wait(rb0, gs0)
                relu_buf(rb0)
                wb_out(i0, rb0)

            @pl.when(i1 < row_chunks)
            def _():
                @pl.when(i1 + n_tiles < row_chunks)
                def _():
                    wb_in(i1 + n_tiles, rb0, gs0)

                wb_in_wait(rb1, gs1)
                relu_buf(rb1)
                wb_out(i1, rb1)

            return carry

        lax.fori_loop(0, -(-wb_iters // 2), wpair, 0)

    return agg(y_flat, gidx, sidx, vals)


def kernel(x_u, x_v, sup_vals, weights_u, weights_v, sup_rows, sup_cols):
    n_u = x_u.shape[0]
    n_v = x_v.shape[0]
    rows = sup_rows.astype(jnp.int32)
    cols = sup_cols.astype(jnp.int32)
    vals = sup_vals.astype(jnp.float32)

    y_v = _compute_y(x_v, weights_v)
    y_v2 = y_v.reshape(-1, _HALF)
    z_u = _sc_agg(y_v2, cols, rows, vals, n_u, n_v)
    y_u = _compute_y(x_u, weights_u)
    y_u2 = y_u.reshape(-1, _HALF)
    z_v = _sc_agg(y_u2, rows, cols, vals, n_v, n_u)
    return z_u, z_v
